# Initial kernel scaffold; baseline (speedup 1.0000x reference)
import jax
import jax.numpy as jnp
from jax.experimental import pallas as pl


def kernel(logits):
    raise NotImplementedError("write your pallas kernel here")


# speedup vs baseline: 3.9758x; 3.9758x over previous
import functools

import jax
import jax.numpy as jnp
import numpy as np
from jax.experimental import pallas as pl

_VOCAB = 64
_TOP_K = 10
_TOP_P = 0.9
_NUM_SAMPLES = 10
_NEG = -1e9
_TINY = np.float32(np.finfo(np.float32).tiny)
_KEY_HI = np.uint32(0)
_KEY_LO = np.uint32(42)
_ROWS_PER_BLOCK = 128


def _rotl(x, d):
    return (x << np.uint32(d)) | (x >> np.uint32(32 - d))


def _threefry_bits(x1):
    ks0, ks1 = _KEY_HI, _KEY_LO
    ks2 = np.uint32(ks0 ^ ks1 ^ np.uint32(0x1BD11BDA))
    ks = (ks0, ks1, ks2)
    x0 = jnp.full(x1.shape, ks0, jnp.uint32)
    x1 = x1 + ks1
    rots = ((13, 15, 26, 6), (17, 29, 16, 24))
    for i in range(5):
        for r in rots[i % 2]:
            x0 = x0 + x1
            x1 = _rotl(x1, r)
            x1 = x0 ^ x1
        x0 = x0 + ks[(i + 1) % 3]
        x1 = x1 + ks[(i + 2) % 3] + np.uint32(i + 1)
    return x0 ^ x1


def _gumbel(flat_idx):
    bits = _threefry_bits(flat_idx.astype(jnp.uint32))
    fb = (bits >> np.uint32(9)) | np.uint32(0x3F800000)
    f = jax.lax.bitcast_convert_type(fb, jnp.float32) - np.float32(1.0)
    u = f * (np.float32(1.0) - _TINY) + _TINY
    u = jnp.maximum(_TINY, u)
    return -jnp.log(-jnp.log(u))


def _ssp_kernel(n_rows, logits_ref, samples_ref, probs_ref):
    rblk = logits_ref.shape[0]
    col = jax.lax.broadcasted_iota(jnp.int32, (rblk, _VOCAB), 1)
    row = jax.lax.broadcasted_iota(jnp.int32, (rblk, _VOCAB), 0)

    x = logits_ref[:, :]
    aa = (col >= 4) & (col < 24)
    x = jnp.where(aa, x, _NEG)

    xj = x[:, :, None]
    xi = x[:, None, :]
    gt = xj > xi
    c = jnp.sum(gt.astype(jnp.float32), axis=1)
    topk_keep = c <= np.float32(_TOP_K - 1)

    y = jnp.where(topk_keep, x, _NEG)
    m = jnp.max(y, axis=-1, keepdims=True)
    q = jnp.exp(y - m)
    p = q / jnp.sum(q, axis=-1, keepdims=True)

    ji = jax.lax.broadcasted_iota(jnp.int32, (_VOCAB, _VOCAB), 0)
    ii = jax.lax.broadcasted_iota(jnp.int32, (_VOCAB, _VOCAB), 1)
    gl = gt | ((xj == xi) & (ji < ii)[None])
    glf = gl.astype(jnp.float32)
    cum = jnp.sum(p[:, :, None] * glf, axis=1) + p
    rank0 = jnp.sum(glf, axis=1) == np.float32(0.0)
    keep = topk_keep & (rank0 | (cum <= np.float32(_TOP_P)))

    zf = jnp.where(keep, x, _NEG)
    m2 = jnp.max(zf, axis=-1, keepdims=True)
    q2 = jnp.exp(zf - m2)
    probs_ref[:, :] = q2 / jnp.sum(q2, axis=-1, keepdims=True)

    base = (pl.program_id(0) * rblk + row) * _VOCAB + col
    for s in range(_NUM_SAMPLES):
        v = zf + _gumbel(base + np.int32(s * n_rows * _VOCAB))
        vmax = jnp.max(v, axis=-1, keepdims=True)
        idx = jnp.min(jnp.where(v == vmax, col, _VOCAB), axis=-1)
        samples_ref[s, :] = idx


@jax.jit
def kernel(logits):
    n_rows = logits.shape[0]
    rblk = _ROWS_PER_BLOCK
    grid = (n_rows // rblk,)
    samples, probs = pl.pallas_call(
        functools.partial(_ssp_kernel, n_rows),
        grid=grid,
        in_specs=[pl.BlockSpec((rblk, _VOCAB), lambda b: (b, 0))],
        out_specs=[
            pl.BlockSpec((_NUM_SAMPLES, rblk), lambda b: (0, b)),
            pl.BlockSpec((rblk, _VOCAB), lambda b: (b, 0)),
        ],
        out_shape=[
            jax.ShapeDtypeStruct((_NUM_SAMPLES, n_rows), jnp.int32),
            jax.ShapeDtypeStruct((n_rows, _VOCAB), jnp.float32),
        ],
    )(logits.astype(jnp.float32))
    return samples, probs


# speedup vs baseline: 9.4503x; 2.3770x over previous
import functools

import jax
import jax.numpy as jnp
import numpy as np
from jax.experimental import pallas as pl

_VOCAB = 64
_TOP_K = 10
_TOP_P = np.float32(0.9)
_NUM_SAMPLES = 10
_NEG = np.float32(-1e9)
_TINY = np.float32(np.finfo(np.float32).tiny)
_KEY_HI = np.uint32(0)
_KEY_LO = np.uint32(42)
_ROWS_PER_BLOCK = 256


def _rotl(x, d):
    return (x << np.uint32(d)) | (x >> np.uint32(32 - d))


def _threefry_bits(x1):
    ks0, ks1 = _KEY_HI, _KEY_LO
    ks2 = np.uint32(ks0 ^ ks1 ^ np.uint32(0x1BD11BDA))
    ks = (ks0, ks1, ks2)
    x0 = jnp.full(x1.shape, ks0, jnp.uint32)
    x1 = x1 + ks1
    rots = ((13, 15, 26, 6), (17, 29, 16, 24))
    for i in range(5):
        for r in rots[i % 2]:
            x0 = x0 + x1
            x1 = _rotl(x1, r)
            x1 = x0 ^ x1
        x0 = x0 + ks[(i + 1) % 3]
        x1 = x1 + ks[(i + 2) % 3] + np.uint32(i + 1)
    return x0 ^ x1


def _gumbel(flat_idx):
    bits = _threefry_bits(flat_idx.astype(jnp.uint32))
    fb = (bits >> np.uint32(9)) | np.uint32(0x3F800000)
    f = jax.lax.bitcast_convert_type(fb, jnp.float32) - np.float32(1.0)
    u = f * (np.float32(1.0) - _TINY) + _TINY
    u = jnp.maximum(_TINY, u)
    return -jnp.log(-jnp.log(u))


def _ssp_kernel(n_rows, logits_ref, samples_ref, probs_ref):
    rblk = logits_ref.shape[0]
    col = jax.lax.broadcasted_iota(jnp.int32, (rblk, _VOCAB), 1)

    x = logits_ref[:, :]
    aa = (col >= 4) & (col < 24)
    x = jnp.where(aa, x, _NEG)

    vals, idxs = [], []
    w = x
    for _ in range(_TOP_K):
        v = jnp.max(w, axis=1, keepdims=True)
        f = jnp.min(jnp.where(w == v, col, _VOCAB), axis=1, keepdims=True)
        w = jnp.where(col == f, _NEG, w)
        vals.append(v)
        idxs.append(f)

    q = [jnp.exp(v - vals[0]) for v in vals]
    z = q[0]
    for t in range(1, _TOP_K):
        z = z + q[t]
    cut_v, cut_i = vals[0], idxs[0]
    cum = q[0] / z
    for t in range(1, _TOP_K):
        cum = cum + q[t] / z
        k = cum <= _TOP_P
        cut_v = jnp.where(k, vals[t], cut_v)
        cut_i = jnp.where(k, idxs[t], cut_i)

    keep = (x > cut_v) | ((x == cut_v) & (col <= cut_i))
    zf = jnp.where(keep, x, _NEG)
    q2 = jnp.exp(zf - vals[0])
    probs_ref[:, :] = q2 / jnp.sum(q2, axis=1, keepdims=True)

    nv = n_rows * _VOCAB
    col2 = jax.lax.broadcasted_iota(jnp.int32, (rblk, 2 * _VOCAB), 1)
    row2 = jax.lax.broadcasted_iota(jnp.int32, (rblk, 2 * _VOCAB), 0)
    lane_off = ((pl.program_id(0) * rblk + row2) * _VOCAB + col2
                + jnp.where(col2 >= _VOCAB, np.int32(5 * nv - _VOCAB), 0))
    zf2 = jnp.concatenate([zf, zf], axis=1)
    for s in range(_NUM_SAMPLES // 2):
        v2 = zf2 + _gumbel(lane_off + np.int32(s * nv))
        for h in range(2):
            v = v2[:, h * _VOCAB:(h + 1) * _VOCAB]
            vmax = jnp.max(v, axis=1, keepdims=True)
            idx = jnp.min(jnp.where(v == vmax, col, _VOCAB), axis=1)
            samples_ref[s + 5 * h, :] = idx


@jax.jit
def kernel(logits):
    n_rows = logits.shape[0]
    rblk = _ROWS_PER_BLOCK
    grid = (n_rows // rblk,)
    samples, probs = pl.pallas_call(
        functools.partial(_ssp_kernel, n_rows),
        grid=grid,
        in_specs=[pl.BlockSpec((rblk, _VOCAB), lambda b: (b, 0))],
        out_specs=[
            pl.BlockSpec((_NUM_SAMPLES, rblk), lambda b: (0, b)),
            pl.BlockSpec((rblk, _VOCAB), lambda b: (b, 0)),
        ],
        out_shape=[
            jax.ShapeDtypeStruct((_NUM_SAMPLES, n_rows), jnp.int32),
            jax.ShapeDtypeStruct((n_rows, _VOCAB), jnp.float32),
        ],
    )(logits.astype(jnp.float32))
    return samples, probs


# speedup vs baseline: 12.7092x; 1.3448x over previous
import functools

import jax
import jax.numpy as jnp
import numpy as np
from jax.experimental import pallas as pl

_VOCAB = 64
_TOP_K = 10
_TOP_P = np.float32(0.9)
_NUM_SAMPLES = 10
_NEG = np.float32(-1e9)
_TINY = np.float32(np.finfo(np.float32).tiny)
_KEY_HI = np.uint32(0)
_KEY_LO = np.uint32(42)
_ROWS_PER_BLOCK = 256


def _rotl(x, d):
    return (x << np.uint32(d)) | (x >> np.uint32(32 - d))


def _threefry_bits(x1):
    ks0, ks1 = _KEY_HI, _KEY_LO
    ks2 = np.uint32(ks0 ^ ks1 ^ np.uint32(0x1BD11BDA))
    ks = (ks0, ks1, ks2)
    x0 = jnp.full(x1.shape, ks0, jnp.uint32)
    x1 = x1 + ks1
    rots = ((13, 15, 26, 6), (17, 29, 16, 24))
    for i in range(5):
        for r in rots[i % 2]:
            x0 = x0 + x1
            x1 = _rotl(x1, r)
            x1 = x0 ^ x1
        x0 = x0 + ks[(i + 1) % 3]
        x1 = x1 + ks[(i + 2) % 3] + np.uint32(i + 1)
    return x0 ^ x1


def _gumbel(flat_idx):
    bits = _threefry_bits(flat_idx.astype(jnp.uint32))
    fb = (bits >> np.uint32(9)) | np.uint32(0x3F800000)
    f = jax.lax.bitcast_convert_type(fb, jnp.float32) - np.float32(1.0)
    u = f * (np.float32(1.0) - _TINY) + _TINY
    u = jnp.maximum(_TINY, u)
    return -jnp.log(-jnp.log(u))


def _ssp_kernel(n_rows, logits_ref, samples_ref, probs_ref):
    rblk = logits_ref.shape[0]
    col = jax.lax.broadcasted_iota(jnp.int32, (rblk, _VOCAB), 1)

    x = logits_ref[:, :]
    aa = (col >= 4) & (col < 24)
    x = jnp.where(aa, x, _NEG)

    vals, idxs = [], []
    w = x
    for _ in range(_TOP_K):
        v = jnp.max(w, axis=1, keepdims=True)
        f = jnp.min(jnp.where(w == v, col, _VOCAB), axis=1, keepdims=True)
        w = jnp.where(col == f, _NEG, w)
        vals.append(v)
        idxs.append(f)

    q = [jnp.exp(v - vals[0]) for v in vals]
    z = q[0]
    for t in range(1, _TOP_K):
        z = z + q[t]
    cut_v, cut_i = vals[0], idxs[0]
    kflags = [None] * _TOP_K
    cum = q[0] / z
    for t in range(1, _TOP_K):
        cum = cum + q[t] / z
        k = cum <= _TOP_P
        kflags[t] = k
        cut_v = jnp.where(k, vals[t], cut_v)
        cut_i = jnp.where(k, idxs[t], cut_i)

    keep = (x > cut_v) | ((x == cut_v) & (col <= cut_i))
    zf = jnp.where(keep, x, _NEG)
    q2 = jnp.exp(zf - vals[0])
    probs_ref[:, :] = q2 / jnp.sum(q2, axis=1, keepdims=True)

    nv = n_rows * _VOCAB
    lane = jax.lax.broadcasted_iota(jnp.int32, (rblk, 128), 1)
    sl = jnp.zeros_like(lane)
    for s in range(1, _NUM_SAMPLES):
        sl = sl + (lane >= 10 * s).astype(jnp.int32)
    tl = lane - 10 * sl
    val_lane = jnp.full((rblk, 128), _NEG, jnp.float32)
    idx_lane = jnp.zeros((rblk, 128), jnp.int32)
    for t in range(_TOP_K):
        m = tl == t
        slot_v = vals[t] if t == 0 else jnp.where(kflags[t], vals[t], _NEG)
        val_lane = jnp.where(m, jnp.broadcast_to(slot_v, (rblk, 128)), val_lane)
        idx_lane = jnp.where(m, jnp.broadcast_to(idxs[t], (rblk, 128)), idx_lane)
    row128 = jax.lax.broadcasted_iota(jnp.int32, (rblk, 128), 0)
    glob_row = pl.program_id(0) * rblk + row128
    f = sl * np.int32(nv) + glob_row * _VOCAB + idx_lane
    v = val_lane + _gumbel(f)
    for s in range(_NUM_SAMPLES):
        vs = jnp.where(sl == s, v, _NEG)
        vmax = jnp.max(vs, axis=1, keepdims=True)
        win = jnp.min(jnp.where(vs == vmax, idx_lane, _VOCAB), axis=1)
        samples_ref[s, :] = win


@jax.jit
def kernel(logits):
    n_rows = logits.shape[0]
    rblk = _ROWS_PER_BLOCK
    grid = (n_rows // rblk,)
    samples, probs = pl.pallas_call(
        functools.partial(_ssp_kernel, n_rows),
        grid=grid,
        in_specs=[pl.BlockSpec((rblk, _VOCAB), lambda b: (b, 0))],
        out_specs=[
            pl.BlockSpec((_NUM_SAMPLES, rblk), lambda b: (0, b)),
            pl.BlockSpec((rblk, _VOCAB), lambda b: (b, 0)),
        ],
        out_shape=[
            jax.ShapeDtypeStruct((_NUM_SAMPLES, n_rows), jnp.int32),
            jax.ShapeDtypeStruct((n_rows, _VOCAB), jnp.float32),
        ],
    )(logits.astype(jnp.float32))
    return samples, probs


# speedup vs baseline: 71.8281x; 5.6517x over previous
import functools

import jax
import jax.numpy as jnp
import numpy as np
from jax.experimental import pallas as pl

_VOCAB = 64
_SLAB = 24
_AA_LO = 4
_TOP_K = 10
_TOP_P = np.float32(0.9)
_NUM_SAMPLES = 10
_NEG = np.float32(-1e9)
_TINY = np.float32(np.finfo(np.float32).tiny)
_KEY_HI = np.uint32(0)
_KEY_LO = np.uint32(42)
_ROWS_PER_BLOCK = 256


def _rotl(x, d):
    return (x << np.uint32(d)) | (x >> np.uint32(32 - d))


def _threefry_bits(x1):
    ks0, ks1 = _KEY_HI, _KEY_LO
    ks2 = np.uint32(ks0 ^ ks1 ^ np.uint32(0x1BD11BDA))
    ks = (ks0, ks1, ks2)
    x0 = jnp.full(x1.shape, ks0, jnp.uint32)
    x1 = x1 + ks1
    rots = ((13, 15, 26, 6), (17, 29, 16, 24))
    for i in range(5):
        for r in rots[i % 2]:
            x0 = x0 + x1
            x1 = _rotl(x1, r)
            x1 = x0 ^ x1
        x0 = x0 + ks[(i + 1) % 3]
        x1 = x1 + ks[(i + 2) % 3] + np.uint32(i + 1)
    return x0 ^ x1


def _gumbel(flat_idx):
    bits = _threefry_bits(flat_idx.astype(jnp.uint32))
    fb = (bits >> np.uint32(9)) | np.uint32(0x3F800000)
    f = jax.lax.bitcast_convert_type(fb, jnp.float32) - np.float32(1.0)
    u = f * (np.float32(1.0) - _TINY) + _TINY
    u = jnp.maximum(_TINY, u)
    return -jnp.log(-jnp.log(u))


def _ssp_kernel(n_rows, xt_ref, samples_ref, probs_ref):
    rblk = xt_ref.shape[1]
    x = xt_ref[0:_SLAB, :]
    voc = jax.lax.broadcasted_iota(jnp.int32, (_SLAB, rblk), 0)

    xm = jnp.where(voc >= _AA_LO, x, _NEG)
    w = xm

    vals, idxs = [], []
    for _ in range(_TOP_K):
        v = jnp.max(w, axis=0, keepdims=True)
        f = jnp.min(jnp.where(w == v, voc, _VOCAB), axis=0, keepdims=True)
        w = jnp.where(voc == f, _NEG, w)
        vals.append(v)
        idxs.append(f)

    q = [jnp.exp(v - vals[0]) for v in vals]
    z = q[0]
    for t in range(1, _TOP_K):
        z = z + q[t]
    cut_v, cut_i = vals[0], idxs[0]
    kflags = [None] * _TOP_K
    cum = q[0] / z
    for t in range(1, _TOP_K):
        cum = cum + q[t] / z
        k = cum <= _TOP_P
        kflags[t] = k
        cut_v = jnp.where(k, vals[t], cut_v)
        cut_i = jnp.where(k, idxs[t], cut_i)

    keep = (xm > cut_v) | ((xm == cut_v) & (voc <= cut_i))
    zf = jnp.where(keep, xm, _NEG)
    q2 = jnp.exp(zf - vals[0])
    probs_ref[0:_SLAB, :] = q2 / jnp.sum(q2, axis=0, keepdims=True)
    probs_ref[_SLAB:_VOCAB, :] = jnp.zeros((_VOCAB - _SLAB, rblk), jnp.float32)

    nv = n_rows * _VOCAB
    s_iota = jax.lax.broadcasted_iota(jnp.int32, (_NUM_SAMPLES, rblk), 0)
    grow = (pl.program_id(0) * rblk
            + jax.lax.broadcasted_iota(jnp.int32, (_NUM_SAMPLES, rblk), 1))
    best_v = None
    best_i = None
    for t in range(_TOP_K):
        slot_v = vals[t] if t == 0 else jnp.where(kflags[t], vals[t], _NEG)
        fidx = s_iota * np.int32(nv) + grow * _VOCAB + idxs[t]
        cand_v = slot_v + _gumbel(fidx)
        cand_i = jnp.broadcast_to(idxs[t], (_NUM_SAMPLES, rblk))
        if t == 0:
            best_v, best_i = cand_v, cand_i
        else:
            sel = (best_v > cand_v) | ((best_v == cand_v) & (best_i <= cand_i))
            best_v = jnp.where(sel, best_v, cand_v)
            best_i = jnp.where(sel, best_i, cand_i)
    samples_ref[:, :] = best_i


@jax.jit
def kernel(logits):
    n_rows = logits.shape[0]
    rblk = _ROWS_PER_BLOCK
    grid = (n_rows // rblk,)
    samples, probs_t = pl.pallas_call(
        functools.partial(_ssp_kernel, n_rows),
        grid=grid,
        in_specs=[pl.BlockSpec((_VOCAB, rblk), lambda b: (0, b))],
        out_specs=[
            pl.BlockSpec((_NUM_SAMPLES, rblk), lambda b: (0, b)),
            pl.BlockSpec((_VOCAB, rblk), lambda b: (0, b)),
        ],
        out_shape=[
            jax.ShapeDtypeStruct((_NUM_SAMPLES, n_rows), jnp.int32),
            jax.ShapeDtypeStruct((_VOCAB, n_rows), jnp.float32),
        ],
    )(logits.astype(jnp.float32).T)
    return samples, probs_t.T


# speedup vs baseline: 84.2533x; 1.1730x over previous
import functools

import jax
import jax.numpy as jnp
import numpy as np
from jax.experimental import pallas as pl

_VOCAB = 64
_SLAB = 24
_AA_LO = 4
_TOP_K = 10
_TOP_P = np.float32(0.9)
_NUM_SAMPLES = 10
_NEG = np.float32(-1e9)
_TINY = np.float32(np.finfo(np.float32).tiny)
_KEY_HI = np.uint32(0)
_KEY_LO = np.uint32(42)
_ROWS_PER_BLOCK = 256


def _rotl(x, d):
    return (x << np.uint32(d)) | (x >> np.uint32(32 - d))


def _threefry_bits(x1):
    ks0, ks1 = _KEY_HI, _KEY_LO
    ks2 = np.uint32(ks0 ^ ks1 ^ np.uint32(0x1BD11BDA))
    ks = (ks0, ks1, ks2)
    x0 = jnp.full(x1.shape, ks0, jnp.uint32)
    x1 = x1 + ks1
    rots = ((13, 15, 26, 6), (17, 29, 16, 24))
    for i in range(5):
        for r in rots[i % 2]:
            x0 = x0 + x1
            x1 = _rotl(x1, r)
            x1 = x0 ^ x1
        x0 = x0 + ks[(i + 1) % 3]
        x1 = x1 + ks[(i + 2) % 3] + np.uint32(i + 1)
    return x0 ^ x1


def _gumbel(flat_idx):
    bits = _threefry_bits(flat_idx.astype(jnp.uint32))
    fb = (bits >> np.uint32(9)) | np.uint32(0x3F800000)
    f = jax.lax.bitcast_convert_type(fb, jnp.float32) - np.float32(1.0)
    u = f * (np.float32(1.0) - _TINY) + _TINY
    u = jnp.maximum(_TINY, u)
    return -jnp.log(-jnp.log(u))


def _ssp_kernel(n_rows, xt_ref, samples_ref, probs_ref):
    rblk = xt_ref.shape[1]
    x = xt_ref[0:_SLAB, :]
    voc = jax.lax.broadcasted_iota(jnp.int32, (_SLAB, rblk), 0)

    xm = jnp.where(voc >= _AA_LO, x, _NEG)
    w = xm

    vals, idxs = [], []
    for _ in range(_TOP_K):
        v = jnp.max(w, axis=0, keepdims=True)
        f = jnp.min(jnp.where(w == v, voc, _VOCAB), axis=0, keepdims=True)
        w = jnp.where(voc == f, _NEG, w)
        vals.append(v)
        idxs.append(f)

    q = [jnp.exp(v - vals[0]) for v in vals]
    z = q[0]
    for t in range(1, _TOP_K):
        z = z + q[t]
    cut_v, cut_i = vals[0], idxs[0]
    kflags = [None] * _TOP_K
    cum = q[0] / z
    for t in range(1, _TOP_K):
        cum = cum + q[t] / z
        k = cum <= _TOP_P
        kflags[t] = k
        cut_v = jnp.where(k, vals[t], cut_v)
        cut_i = jnp.where(k, idxs[t], cut_i)

    keep = (xm > cut_v) | ((xm == cut_v) & (voc <= cut_i))
    zf = jnp.where(keep, xm, _NEG)
    q2 = jnp.exp(zf - vals[0])
    probs_ref[0:_SLAB, :] = q2 / jnp.sum(q2, axis=0, keepdims=True)
    probs_ref[_SLAB:_VOCAB, :] = jnp.zeros((_VOCAB - _SLAB, rblk), jnp.float32)

    nv = n_rows * _VOCAB
    grow1 = (pl.program_id(0) * rblk
             + jax.lax.broadcasted_iota(jnp.int32, (1, rblk), 1))
    s_off = (jax.lax.broadcasted_iota(jnp.int32, (_NUM_SAMPLES, rblk), 0)
             * np.int32(nv))
    parts = [s_off + (grow1 * _VOCAB + idxs[t]) for t in range(_TOP_K)]
    gum_all = _gumbel(jnp.concatenate(parts, axis=0))
    best_v = None
    best_i = None
    for t in range(_TOP_K):
        slot_v = vals[t] if t == 0 else jnp.where(kflags[t], vals[t], _NEG)
        g = jax.lax.slice_in_dim(
            gum_all, t * _NUM_SAMPLES, (t + 1) * _NUM_SAMPLES, axis=0)
        cand_v = slot_v + g
        cand_i = jnp.broadcast_to(idxs[t], (_NUM_SAMPLES, rblk))
        if t == 0:
            best_v, best_i = cand_v, cand_i
        else:
            sel = (best_v > cand_v) | ((best_v == cand_v) & (best_i <= cand_i))
            best_v = jnp.where(sel, best_v, cand_v)
            best_i = jnp.where(sel, best_i, cand_i)
    samples_ref[:, :] = best_i


@jax.jit
def kernel(logits):
    n_rows = logits.shape[0]
    rblk = _ROWS_PER_BLOCK
    grid = (n_rows // rblk,)
    samples, probs_t = pl.pallas_call(
        functools.partial(_ssp_kernel, n_rows),
        grid=grid,
        in_specs=[pl.BlockSpec((_VOCAB, rblk), lambda b: (0, b))],
        out_specs=[
            pl.BlockSpec((_NUM_SAMPLES, rblk), lambda b: (0, b)),
            pl.BlockSpec((_VOCAB, rblk), lambda b: (0, b)),
        ],
        out_shape=[
            jax.ShapeDtypeStruct((_NUM_SAMPLES, n_rows), jnp.int32),
            jax.ShapeDtypeStruct((_VOCAB, n_rows), jnp.float32),
        ],
    )(logits.astype(jnp.float32).T)
    return samples, probs_t.T


# speedup vs baseline: 84.4390x; 1.0022x over previous
import functools

import jax
import jax.numpy as jnp
import numpy as np
from jax.experimental import pallas as pl
from jax.experimental.pallas import tpu as pltpu

_VOCAB = 64
_SLAB = 24
_AA_LO = 4
_TOP_K = 10
_TOP_P = np.float32(0.9)
_NUM_SAMPLES = 10
_NEG = np.float32(-1e9)
_TINY = np.float32(np.finfo(np.float32).tiny)
_KEY_HI = np.uint32(0)
_KEY_LO = np.uint32(42)
_ROWS_PER_BLOCK = 256


def _rotl(x, d):
    return (x << np.uint32(d)) | (x >> np.uint32(32 - d))


def _threefry_bits(x1):
    ks0, ks1 = _KEY_HI, _KEY_LO
    ks2 = np.uint32(ks0 ^ ks1 ^ np.uint32(0x1BD11BDA))
    ks = (ks0, ks1, ks2)
    x0 = jnp.full(x1.shape, ks0, jnp.uint32)
    x1 = x1 + ks1
    rots = ((13, 15, 26, 6), (17, 29, 16, 24))
    for i in range(5):
        for r in rots[i % 2]:
            x0 = x0 + x1
            x1 = _rotl(x1, r)
            x1 = x0 ^ x1
        x0 = x0 + ks[(i + 1) % 3]
        x1 = x1 + ks[(i + 2) % 3] + np.uint32(i + 1)
    return x0 ^ x1


def _gumbel(flat_idx):
    bits = _threefry_bits(flat_idx.astype(jnp.uint32))
    fb = (bits >> np.uint32(9)) | np.uint32(0x3F800000)
    f = jax.lax.bitcast_convert_type(fb, jnp.float32) - np.float32(1.0)
    u = f * (np.float32(1.0) - _TINY) + _TINY
    u = jnp.maximum(_TINY, u)
    return -jnp.log(-jnp.log(u))


def _ssp_kernel(n_rows, xt_ref, samples_ref, probs_ref):
    rblk = xt_ref.shape[1]
    x = xt_ref[0:_SLAB, :]
    voc = jax.lax.broadcasted_iota(jnp.int32, (_SLAB, rblk), 0)

    xm = jnp.where(voc >= _AA_LO, x, _NEG)
    w = xm

    vals, idxs = [], []
    for _ in range(_TOP_K):
        v = jnp.max(w, axis=0, keepdims=True)
        f = jnp.min(jnp.where(w == v, voc, _VOCAB), axis=0, keepdims=True)
        w = jnp.where(voc == f, _NEG, w)
        vals.append(v)
        idxs.append(f)

    q = [jnp.exp(v - vals[0]) for v in vals]
    z = q[0]
    for t in range(1, _TOP_K):
        z = z + q[t]
    cut_v, cut_i = vals[0], idxs[0]
    kflags = [None] * _TOP_K
    cum = q[0] / z
    for t in range(1, _TOP_K):
        cum = cum + q[t] / z
        k = cum <= _TOP_P
        kflags[t] = k
        cut_v = jnp.where(k, vals[t], cut_v)
        cut_i = jnp.where(k, idxs[t], cut_i)

    keep = (xm > cut_v) | ((xm == cut_v) & (voc <= cut_i))
    zf = jnp.where(keep, xm, _NEG)
    q2 = jnp.exp(zf - vals[0])
    probs_ref[0:_SLAB, :] = q2 / jnp.sum(q2, axis=0, keepdims=True)
    probs_ref[_SLAB:_VOCAB, :] = jnp.zeros((_VOCAB - _SLAB, rblk), jnp.float32)

    nv = n_rows * _VOCAB
    grow1 = (pl.program_id(0) * rblk
             + jax.lax.broadcasted_iota(jnp.int32, (1, rblk), 1))
    s_off = (jax.lax.broadcasted_iota(jnp.int32, (_NUM_SAMPLES, rblk), 0)
             * np.int32(nv))
    parts = [s_off + (grow1 * _VOCAB + idxs[t]) for t in range(_TOP_K)]
    gum_all = _gumbel(jnp.concatenate(parts, axis=0))
    best_v = None
    best_i = None
    for t in range(_TOP_K):
        slot_v = vals[t] if t == 0 else jnp.where(kflags[t], vals[t], _NEG)
        g = jax.lax.slice_in_dim(
            gum_all, t * _NUM_SAMPLES, (t + 1) * _NUM_SAMPLES, axis=0)
        cand_v = slot_v + g
        cand_i = jnp.broadcast_to(idxs[t], (_NUM_SAMPLES, rblk))
        if t == 0:
            best_v, best_i = cand_v, cand_i
        else:
            sel = (best_v > cand_v) | ((best_v == cand_v) & (best_i <= cand_i))
            best_v = jnp.where(sel, best_v, cand_v)
            best_i = jnp.where(sel, best_i, cand_i)
    samples_ref[:, :] = best_i


@jax.jit
def kernel(logits):
    n_rows = logits.shape[0]
    rblk = _ROWS_PER_BLOCK
    grid = (n_rows // rblk,)
    samples, probs_t = pl.pallas_call(
        functools.partial(_ssp_kernel, n_rows),
        grid=grid,
        in_specs=[pl.BlockSpec((_VOCAB, rblk), lambda b: (0, b))],
        out_specs=[
            pl.BlockSpec((_NUM_SAMPLES, rblk), lambda b: (0, b)),
            pl.BlockSpec((_VOCAB, rblk), lambda b: (0, b)),
        ],
        out_shape=[
            jax.ShapeDtypeStruct((_NUM_SAMPLES, n_rows), jnp.int32),
            jax.ShapeDtypeStruct((_VOCAB, n_rows), jnp.float32),
        ],
        compiler_params=pltpu.CompilerParams(
            dimension_semantics=("parallel",)),
    )(logits.astype(jnp.float32).T)
    return samples, probs_t.T


# speedup vs baseline: 102.7847x; 1.2173x over previous
import functools

import jax
import jax.numpy as jnp
import numpy as np
from jax.experimental import pallas as pl
from jax.experimental.pallas import tpu as pltpu

_VOCAB = 64
_SLAB = 24
_AA_LO = 4
_TOP_K = 10
_TOP_P = np.float32(0.9)
_NUM_SAMPLES = 10
_NEG = np.float32(-1e9)
_TINY = np.float32(np.finfo(np.float32).tiny)
_KEY_HI = np.uint32(0)
_KEY_LO = np.uint32(42)
_ROWS_PER_BLOCK = 512


def _rotl(x, d):
    return (x << np.uint32(d)) | (x >> np.uint32(32 - d))


def _threefry_bits(x1):
    ks0, ks1 = _KEY_HI, _KEY_LO
    ks2 = np.uint32(ks0 ^ ks1 ^ np.uint32(0x1BD11BDA))
    ks = (ks0, ks1, ks2)
    x0 = jnp.full(x1.shape, ks0, jnp.uint32)
    x1 = x1 + ks1
    rots = ((13, 15, 26, 6), (17, 29, 16, 24))
    for i in range(5):
        for r in rots[i % 2]:
            x0 = x0 + x1
            x1 = _rotl(x1, r)
            x1 = x0 ^ x1
        x0 = x0 + ks[(i + 1) % 3]
        x1 = x1 + ks[(i + 2) % 3] + np.uint32(i + 1)
    return x0 ^ x1


def _gumbel(flat_idx):
    bits = _threefry_bits(flat_idx.astype(jnp.uint32))
    fb = (bits >> np.uint32(9)) | np.uint32(0x3F800000)
    f = jax.lax.bitcast_convert_type(fb, jnp.float32) - np.float32(1.0)
    u = f * (np.float32(1.0) - _TINY) + _TINY
    u = jnp.maximum(_TINY, u)
    return -jnp.log(-jnp.log(u))


def _ssp_kernel(n_rows, xt_ref, samples_ref, probs_ref):
    rblk = xt_ref.shape[1]
    x = xt_ref[0:_SLAB, :]
    voc = jax.lax.broadcasted_iota(jnp.int32, (_SLAB, rblk), 0)

    xm = jnp.where(voc >= _AA_LO, x, _NEG)
    w = xm

    vals, idxs = [], []
    for _ in range(_TOP_K):
        v = jnp.max(w, axis=0, keepdims=True)
        f = jnp.min(jnp.where(w == v, voc, _VOCAB), axis=0, keepdims=True)
        w = jnp.where(voc == f, _NEG, w)
        vals.append(v)
        idxs.append(f)

    q = [jnp.exp(v - vals[0]) for v in vals]
    z = q[0]
    for t in range(1, _TOP_K):
        z = z + q[t]
    cut_v, cut_i = vals[0], idxs[0]
    kflags = [None] * _TOP_K
    cum = q[0] / z
    for t in range(1, _TOP_K):
        cum = cum + q[t] / z
        k = cum <= _TOP_P
        kflags[t] = k
        cut_v = jnp.where(k, vals[t], cut_v)
        cut_i = jnp.where(k, idxs[t], cut_i)

    keep = (xm > cut_v) | ((xm == cut_v) & (voc <= cut_i))
    zf = jnp.where(keep, xm, _NEG)
    q2 = jnp.exp(zf - vals[0])
    probs_ref[0:_SLAB, :] = q2 / jnp.sum(q2, axis=0, keepdims=True)
    probs_ref[_SLAB:_VOCAB, :] = jnp.zeros((_VOCAB - _SLAB, rblk), jnp.float32)

    nv = n_rows * _VOCAB
    grow1 = (pl.program_id(0) * rblk
             + jax.lax.broadcasted_iota(jnp.int32, (1, rblk), 1))
    s_off = (jax.lax.broadcasted_iota(jnp.int32, (_NUM_SAMPLES, rblk), 0)
             * np.int32(nv))
    parts = [s_off + (grow1 * _VOCAB + idxs[t]) for t in range(_TOP_K)]
    gum_all = _gumbel(jnp.concatenate(parts, axis=0))
    best_v = None
    best_i = None
    for t in range(_TOP_K):
        slot_v = vals[t] if t == 0 else jnp.where(kflags[t], vals[t], _NEG)
        g = jax.lax.slice_in_dim(
            gum_all, t * _NUM_SAMPLES, (t + 1) * _NUM_SAMPLES, axis=0)
        cand_v = slot_v + g
        cand_i = jnp.broadcast_to(idxs[t], (_NUM_SAMPLES, rblk))
        if t == 0:
            best_v, best_i = cand_v, cand_i
        else:
            sel = (best_v > cand_v) | ((best_v == cand_v) & (best_i <= cand_i))
            best_v = jnp.where(sel, best_v, cand_v)
            best_i = jnp.where(sel, best_i, cand_i)
    samples_ref[:, :] = best_i


@jax.jit
def kernel(logits):
    n_rows = logits.shape[0]
    rblk = _ROWS_PER_BLOCK
    grid = (n_rows // rblk,)
    samples, probs_t = pl.pallas_call(
        functools.partial(_ssp_kernel, n_rows),
        grid=grid,
        in_specs=[pl.BlockSpec((_VOCAB, rblk), lambda b: (0, b))],
        out_specs=[
            pl.BlockSpec((_NUM_SAMPLES, rblk), lambda b: (0, b)),
            pl.BlockSpec((_VOCAB, rblk), lambda b: (0, b)),
        ],
        out_shape=[
            jax.ShapeDtypeStruct((_NUM_SAMPLES, n_rows), jnp.int32),
            jax.ShapeDtypeStruct((_VOCAB, n_rows), jnp.float32),
        ],
        compiler_params=pltpu.CompilerParams(
            dimension_semantics=("parallel",)),
    )(logits.astype(jnp.float32).T)
    return samples, probs_t.T


# speedup vs baseline: 104.5712x; 1.0174x over previous
import functools

import jax
import jax.numpy as jnp
import numpy as np
from jax.experimental import pallas as pl
from jax.experimental.pallas import tpu as pltpu

_VOCAB = 64
_SLAB = 24
_AA_LO = 4
_TOP_K = 10
_TOP_P = np.float32(0.9)
_NUM_SAMPLES = 10
_NEG = np.float32(-1e9)
_TINY = np.float32(np.finfo(np.float32).tiny)
_KEY_HI = np.uint32(0)
_KEY_LO = np.uint32(42)
_ROWS_PER_BLOCK = 1024


def _rotl(x, d):
    return (x << np.uint32(d)) | (x >> np.uint32(32 - d))


def _threefry_bits(x1):
    ks0, ks1 = _KEY_HI, _KEY_LO
    ks2 = np.uint32(ks0 ^ ks1 ^ np.uint32(0x1BD11BDA))
    ks = (ks0, ks1, ks2)
    x0 = jnp.full(x1.shape, ks0, jnp.uint32)
    x1 = x1 + ks1
    rots = ((13, 15, 26, 6), (17, 29, 16, 24))
    for i in range(5):
        for r in rots[i % 2]:
            x0 = x0 + x1
            x1 = _rotl(x1, r)
            x1 = x0 ^ x1
        x0 = x0 + ks[(i + 1) % 3]
        x1 = x1 + ks[(i + 2) % 3] + np.uint32(i + 1)
    return x0 ^ x1


def _gumbel(flat_idx):
    bits = _threefry_bits(flat_idx.astype(jnp.uint32))
    fb = (bits >> np.uint32(9)) | np.uint32(0x3F800000)
    f = jax.lax.bitcast_convert_type(fb, jnp.float32) - np.float32(1.0)
    u = f * (np.float32(1.0) - _TINY) + _TINY
    u = jnp.maximum(_TINY, u)
    return -jnp.log(-jnp.log(u))


def _ssp_kernel(n_rows, xt_ref, samples_ref, probs_ref):
    rblk = xt_ref.shape[1]
    x = xt_ref[0:_SLAB, :]
    voc = jax.lax.broadcasted_iota(jnp.int32, (_SLAB, rblk), 0)

    xm = jnp.where(voc >= _AA_LO, x, _NEG)
    w = xm

    vals, idxs = [], []
    for _ in range(_TOP_K):
        v = jnp.max(w, axis=0, keepdims=True)
        f = jnp.min(jnp.where(w == v, voc, _VOCAB), axis=0, keepdims=True)
        w = jnp.where(voc == f, _NEG, w)
        vals.append(v)
        idxs.append(f)

    q = [jnp.exp(v - vals[0]) for v in vals]
    z = q[0]
    for t in range(1, _TOP_K):
        z = z + q[t]
    cut_v, cut_i = vals[0], idxs[0]
    kflags = [None] * _TOP_K
    cum = q[0] / z
    for t in range(1, _TOP_K):
        cum = cum + q[t] / z
        k = cum <= _TOP_P
        kflags[t] = k
        cut_v = jnp.where(k, vals[t], cut_v)
        cut_i = jnp.where(k, idxs[t], cut_i)

    keep = (xm > cut_v) | ((xm == cut_v) & (voc <= cut_i))
    zf = jnp.where(keep, xm, _NEG)
    q2 = jnp.exp(zf - vals[0])
    probs_ref[0:_SLAB, :] = q2 / jnp.sum(q2, axis=0, keepdims=True)
    probs_ref[_SLAB:_VOCAB, :] = jnp.zeros((_VOCAB - _SLAB, rblk), jnp.float32)

    nv = n_rows * _VOCAB
    grow1 = (pl.program_id(0) * rblk
             + jax.lax.broadcasted_iota(jnp.int32, (1, rblk), 1))
    s_off = (jax.lax.broadcasted_iota(jnp.int32, (_NUM_SAMPLES, rblk), 0)
             * np.int32(nv))
    parts = [s_off + (grow1 * _VOCAB + idxs[t]) for t in range(_TOP_K)]
    gum_all = _gumbel(jnp.concatenate(parts, axis=0))
    best_v = None
    best_i = None
    for t in range(_TOP_K):
        slot_v = vals[t] if t == 0 else jnp.where(kflags[t], vals[t], _NEG)
        g = jax.lax.slice_in_dim(
            gum_all, t * _NUM_SAMPLES, (t + 1) * _NUM_SAMPLES, axis=0)
        cand_v = slot_v + g
        cand_i = jnp.broadcast_to(idxs[t], (_NUM_SAMPLES, rblk))
        if t == 0:
            best_v, best_i = cand_v, cand_i
        else:
            sel = (best_v > cand_v) | ((best_v == cand_v) & (best_i <= cand_i))
            best_v = jnp.where(sel, best_v, cand_v)
            best_i = jnp.where(sel, best_i, cand_i)
    samples_ref[:, :] = best_i


@jax.jit
def kernel(logits):
    n_rows = logits.shape[0]
    rblk = _ROWS_PER_BLOCK
    grid = (n_rows // rblk,)
    samples, probs_t = pl.pallas_call(
        functools.partial(_ssp_kernel, n_rows),
        grid=grid,
        in_specs=[pl.BlockSpec((_VOCAB, rblk), lambda b: (0, b))],
        out_specs=[
            pl.BlockSpec((_NUM_SAMPLES, rblk), lambda b: (0, b)),
            pl.BlockSpec((_VOCAB, rblk), lambda b: (0, b)),
        ],
        out_shape=[
            jax.ShapeDtypeStruct((_NUM_SAMPLES, n_rows), jnp.int32),
            jax.ShapeDtypeStruct((_VOCAB, n_rows), jnp.float32),
        ],
        compiler_params=pltpu.CompilerParams(
            dimension_semantics=("parallel",)),
    )(logits.astype(jnp.float32).T)
    return samples, probs_t.T


# speedup vs baseline: 105.4261x; 1.0082x over previous
import functools

import jax
import jax.numpy as jnp
import numpy as np
from jax.experimental import pallas as pl
from jax.experimental.pallas import tpu as pltpu

_VOCAB = 64
_SLAB = 24
_AA_LO = 4
_TOP_K = 10
_TOP_P = np.float32(0.9)
_NUM_SAMPLES = 10
_NEG = np.float32(-1e9)
_TINY = np.float32(np.finfo(np.float32).tiny)
_KEY_HI = np.uint32(0)
_KEY_LO = np.uint32(42)
_ROWS_PER_BLOCK = 2048


def _rotl(x, d):
    return (x << np.uint32(d)) | (x >> np.uint32(32 - d))


def _threefry_bits(x1):
    ks0, ks1 = _KEY_HI, _KEY_LO
    ks2 = np.uint32(ks0 ^ ks1 ^ np.uint32(0x1BD11BDA))
    ks = (ks0, ks1, ks2)
    x0 = jnp.full(x1.shape, ks0, jnp.uint32)
    x1 = x1 + ks1
    rots = ((13, 15, 26, 6), (17, 29, 16, 24))
    for i in range(5):
        for r in rots[i % 2]:
            x0 = x0 + x1
            x1 = _rotl(x1, r)
            x1 = x0 ^ x1
        x0 = x0 + ks[(i + 1) % 3]
        x1 = x1 + ks[(i + 2) % 3] + np.uint32(i + 1)
    return x0 ^ x1


def _gumbel(flat_idx):
    bits = _threefry_bits(flat_idx.astype(jnp.uint32))
    fb = (bits >> np.uint32(9)) | np.uint32(0x3F800000)
    f = jax.lax.bitcast_convert_type(fb, jnp.float32) - np.float32(1.0)
    u = f * (np.float32(1.0) - _TINY) + _TINY
    u = jnp.maximum(_TINY, u)
    return -jnp.log(-jnp.log(u))


def _ssp_kernel(n_rows, xt_ref, samples_ref, probs_ref):
    rblk = xt_ref.shape[1]
    x = xt_ref[0:_SLAB, :]
    voc = jax.lax.broadcasted_iota(jnp.int32, (_SLAB, rblk), 0)

    xm = jnp.where(voc >= _AA_LO, x, _NEG)
    w = xm

    vals, idxs = [], []
    for _ in range(_TOP_K):
        v = jnp.max(w, axis=0, keepdims=True)
        f = jnp.min(jnp.where(w == v, voc, _VOCAB), axis=0, keepdims=True)
        w = jnp.where(voc == f, _NEG, w)
        vals.append(v)
        idxs.append(f)

    q = [jnp.exp(v - vals[0]) for v in vals]
    z = q[0]
    for t in range(1, _TOP_K):
        z = z + q[t]
    cut_v, cut_i = vals[0], idxs[0]
    kflags = [None] * _TOP_K
    cum = q[0] / z
    for t in range(1, _TOP_K):
        cum = cum + q[t] / z
        k = cum <= _TOP_P
        kflags[t] = k
        cut_v = jnp.where(k, vals[t], cut_v)
        cut_i = jnp.where(k, idxs[t], cut_i)

    keep = (xm > cut_v) | ((xm == cut_v) & (voc <= cut_i))
    zf = jnp.where(keep, xm, _NEG)
    q2 = jnp.exp(zf - vals[0])
    probs_ref[0:_SLAB, :] = q2 / jnp.sum(q2, axis=0, keepdims=True)
    probs_ref[_SLAB:_VOCAB, :] = jnp.zeros((_VOCAB - _SLAB, rblk), jnp.float32)

    nv = n_rows * _VOCAB
    grow1 = (pl.program_id(0) * rblk
             + jax.lax.broadcasted_iota(jnp.int32, (1, rblk), 1))
    s_off = (jax.lax.broadcasted_iota(jnp.int32, (_NUM_SAMPLES, rblk), 0)
             * np.int32(nv))
    parts = [s_off + (grow1 * _VOCAB + idxs[t]) for t in range(_TOP_K)]
    gum_all = _gumbel(jnp.concatenate(parts, axis=0))
    best_v = None
    best_i = None
    for t in range(_TOP_K):
        slot_v = vals[t] if t == 0 else jnp.where(kflags[t], vals[t], _NEG)
        g = jax.lax.slice_in_dim(
            gum_all, t * _NUM_SAMPLES, (t + 1) * _NUM_SAMPLES, axis=0)
        cand_v = slot_v + g
        cand_i = jnp.broadcast_to(idxs[t], (_NUM_SAMPLES, rblk))
        if t == 0:
            best_v, best_i = cand_v, cand_i
        else:
            sel = (best_v > cand_v) | ((best_v == cand_v) & (best_i <= cand_i))
            best_v = jnp.where(sel, best_v, cand_v)
            best_i = jnp.where(sel, best_i, cand_i)
    samples_ref[:, :] = best_i


@jax.jit
def kernel(logits):
    n_rows = logits.shape[0]
    rblk = _ROWS_PER_BLOCK
    grid = (n_rows // rblk,)
    samples, probs_t = pl.pallas_call(
        functools.partial(_ssp_kernel, n_rows),
        grid=grid,
        in_specs=[pl.BlockSpec((_VOCAB, rblk), lambda b: (0, b))],
        out_specs=[
            pl.BlockSpec((_NUM_SAMPLES, rblk), lambda b: (0, b)),
            pl.BlockSpec((_VOCAB, rblk), lambda b: (0, b)),
        ],
        out_shape=[
            jax.ShapeDtypeStruct((_NUM_SAMPLES, n_rows), jnp.int32),
            jax.ShapeDtypeStruct((_VOCAB, n_rows), jnp.float32),
        ],
        compiler_params=pltpu.CompilerParams(
            dimension_semantics=("parallel",)),
    )(logits.astype(jnp.float32).T)
    return samples, probs_t.T


# speedup vs baseline: 105.5473x; 1.0011x over previous
import functools

import jax
import jax.numpy as jnp
import numpy as np
from jax.experimental import pallas as pl
from jax.experimental.pallas import tpu as pltpu

_VOCAB = 64
_SLAB = 24
_AA_LO = 4
_TOP_K = 10
_TOP_P = np.float32(0.9)
_NUM_SAMPLES = 10
_NEG = np.float32(-1e9)
_TINY = np.float32(np.finfo(np.float32).tiny)
_KEY_HI = np.uint32(0)
_KEY_LO = np.uint32(42)
_ROWS_PER_BLOCK = 4096


def _rotl(x, d):
    return (x << np.uint32(d)) | (x >> np.uint32(32 - d))


def _threefry_bits(x1):
    ks0, ks1 = _KEY_HI, _KEY_LO
    ks2 = np.uint32(ks0 ^ ks1 ^ np.uint32(0x1BD11BDA))
    ks = (ks0, ks1, ks2)
    x0 = jnp.full(x1.shape, ks0, jnp.uint32)
    x1 = x1 + ks1
    rots = ((13, 15, 26, 6), (17, 29, 16, 24))
    for i in range(5):
        for r in rots[i % 2]:
            x0 = x0 + x1
            x1 = _rotl(x1, r)
            x1 = x0 ^ x1
        x0 = x0 + ks[(i + 1) % 3]
        x1 = x1 + ks[(i + 2) % 3] + np.uint32(i + 1)
    return x0 ^ x1


def _gumbel(flat_idx):
    bits = _threefry_bits(flat_idx.astype(jnp.uint32))
    fb = (bits >> np.uint32(9)) | np.uint32(0x3F800000)
    f = jax.lax.bitcast_convert_type(fb, jnp.float32) - np.float32(1.0)
    u = f * (np.float32(1.0) - _TINY) + _TINY
    u = jnp.maximum(_TINY, u)
    return -jnp.log(-jnp.log(u))


def _ssp_kernel(n_rows, xt_ref, samples_ref, probs_ref):
    rblk = xt_ref.shape[1]
    x = xt_ref[0:_SLAB, :]
    voc = jax.lax.broadcasted_iota(jnp.int32, (_SLAB, rblk), 0)

    xm = jnp.where(voc >= _AA_LO, x, _NEG)
    w = xm

    vals, idxs = [], []
    for _ in range(_TOP_K):
        v = jnp.max(w, axis=0, keepdims=True)
        f = jnp.min(jnp.where(w == v, voc, _VOCAB), axis=0, keepdims=True)
        w = jnp.where(voc == f, _NEG, w)
        vals.append(v)
        idxs.append(f)

    q = [jnp.exp(v - vals[0]) for v in vals]
    z = q[0]
    for t in range(1, _TOP_K):
        z = z + q[t]
    cut_v, cut_i = vals[0], idxs[0]
    kflags = [None] * _TOP_K
    cum = q[0] / z
    for t in range(1, _TOP_K):
        cum = cum + q[t] / z
        k = cum <= _TOP_P
        kflags[t] = k
        cut_v = jnp.where(k, vals[t], cut_v)
        cut_i = jnp.where(k, idxs[t], cut_i)

    keep = (xm > cut_v) | ((xm == cut_v) & (voc <= cut_i))
    zf = jnp.where(keep, xm, _NEG)
    q2 = jnp.exp(zf - vals[0])
    probs_ref[0:_SLAB, :] = q2 / jnp.sum(q2, axis=0, keepdims=True)
    probs_ref[_SLAB:_VOCAB, :] = jnp.zeros((_VOCAB - _SLAB, rblk), jnp.float32)

    nv = n_rows * _VOCAB
    grow1 = (pl.program_id(0) * rblk
             + jax.lax.broadcasted_iota(jnp.int32, (1, rblk), 1))
    s_off = (jax.lax.broadcasted_iota(jnp.int32, (_NUM_SAMPLES, rblk), 0)
             * np.int32(nv))
    parts = [s_off + (grow1 * _VOCAB + idxs[t]) for t in range(_TOP_K)]
    gum_all = _gumbel(jnp.concatenate(parts, axis=0))
    best_v = None
    best_i = None
    for t in range(_TOP_K):
        slot_v = vals[t] if t == 0 else jnp.where(kflags[t], vals[t], _NEG)
        g = jax.lax.slice_in_dim(
            gum_all, t * _NUM_SAMPLES, (t + 1) * _NUM_SAMPLES, axis=0)
        cand_v = slot_v + g
        cand_i = jnp.broadcast_to(idxs[t], (_NUM_SAMPLES, rblk))
        if t == 0:
            best_v, best_i = cand_v, cand_i
        else:
            sel = (best_v > cand_v) | ((best_v == cand_v) & (best_i <= cand_i))
            best_v = jnp.where(sel, best_v, cand_v)
            best_i = jnp.where(sel, best_i, cand_i)
    samples_ref[:, :] = best_i


@jax.jit
def kernel(logits):
    n_rows = logits.shape[0]
    rblk = _ROWS_PER_BLOCK
    grid = (n_rows // rblk,)
    samples, probs_t = pl.pallas_call(
        functools.partial(_ssp_kernel, n_rows),
        grid=grid,
        in_specs=[pl.BlockSpec((_VOCAB, rblk), lambda b: (0, b))],
        out_specs=[
            pl.BlockSpec((_NUM_SAMPLES, rblk), lambda b: (0, b)),
            pl.BlockSpec((_VOCAB, rblk), lambda b: (0, b)),
        ],
        out_shape=[
            jax.ShapeDtypeStruct((_NUM_SAMPLES, n_rows), jnp.int32),
            jax.ShapeDtypeStruct((_VOCAB, n_rows), jnp.float32),
        ],
        compiler_params=pltpu.CompilerParams(
            dimension_semantics=("parallel",)),
    )(logits.astype(jnp.float32).T)
    return samples, probs_t.T
